# baseline (speedup 1.0000x reference)
import functools

import jax
import jax.numpy as jnp
from jax import lax
from jax.experimental import pallas as pl
from jax.experimental.pallas import tpu as pltpu
from jax.experimental.pallas import tpu_sc as plsc

N = 10000
E = 320000
D = 128
BN_EPS = 1e-3

NC = 2
NS = 16
NW = NC * NS
CHUNK = 128
K = 8 * (-(-E // (NW * CHUNK * 8)))
EPW = K * CHUNK
EPAD = NW * EPW
NPAD = 10240
ZR = 16


def _sc_agg_body(
    h_hbm, srcr_hbm, dstr_hbm, out_hbm, sidx, didx, rows0, rows1, zbuf, acc, sem0, sem1
):
    c = lax.axis_index("c")
    s = lax.axis_index("s")
    w = c * NS + s

    zeros16 = jnp.zeros((16,), jnp.float32)
    for r in range(ZR):
        for q in range(D // 16):
            zbuf[r, pl.ds(q * 16, 16)] = zeros16

    rows_per_tile = NPAD // NS

    def zero_body(t, carry):
        pltpu.sync_copy(zbuf, acc.at[pl.ds(s * rows_per_tile + t * ZR, ZR)])
        return carry

    lax.fori_loop(0, rows_per_tile // ZR, zero_body, 0)

    plsc.subcore_barrier()

    NH = K // 2

    for h in range(2):
        pltpu.sync_copy(srcr_hbm.at[pl.ds(w * K + h * NH, NH)], sidx)
        pltpu.sync_copy(dstr_hbm.at[pl.ds(w * K + h * NH, NH)], didx)
        pltpu.async_copy(h_hbm.at[sidx.at[0]], rows0, sem0)

        def pair_body(t, carry):
            e0 = 2 * t
            pltpu.make_async_copy(h_hbm.at[sidx.at[e0]], rows0, sem0).wait()
            pltpu.async_copy(h_hbm.at[sidx.at[e0 + 1]], rows1, sem1)
            pltpu.sync_copy(rows0, acc.at[didx.at[e0]], add=True)
            nxt = jnp.minimum(e0 + 2, NH - 1)
            pltpu.make_async_copy(h_hbm.at[sidx.at[e0 + 1]], rows1, sem1).wait()
            pltpu.async_copy(h_hbm.at[sidx.at[nxt]], rows0, sem0)
            pltpu.sync_copy(rows1, acc.at[didx.at[e0 + 1]], add=True)
            return carry

        lax.fori_loop(0, NH // 2, pair_body, 0)
        pltpu.make_async_copy(h_hbm.at[sidx.at[NH - 1]], rows0, sem0).wait()

    plsc.subcore_barrier()

    pltpu.sync_copy(
        acc.at[pl.ds(s * rows_per_tile, rows_per_tile)],
        out_hbm.at[pl.ds(c * NPAD + s * rows_per_tile, rows_per_tile)],
    )


def _make_sc_agg():
    mesh = plsc.VectorSubcoreMesh(
        core_axis_name="c", subcore_axis_name="s", num_cores=NC, num_subcores=NS
    )
    return pl.kernel(
        _sc_agg_body,
        out_type=jax.ShapeDtypeStruct((NC * NPAD, D), jnp.float32),
        mesh=mesh,
        scratch_types=[
            pltpu.VMEM((K // 2, CHUNK), jnp.int32),
            pltpu.VMEM((K // 2, CHUNK), jnp.int32),
            pltpu.VMEM((CHUNK, D), jnp.float32),
            pltpu.VMEM((CHUNK, D), jnp.float32),
            pltpu.VMEM((ZR, D), jnp.float32),
            pltpu.VMEM_SHARED((NPAD, D), jnp.float32),
            pltpu.SemaphoreType.DMA,
            pltpu.SemaphoreType.DMA,
        ],
    )


_SQRT_HALF = 0.7071067811865476


def _ffn_body(h_ref, p0_ref, p1_ref, w_ref, b_ref, o_ref):
    hsum = h_ref[...] + p0_ref[...] + p1_ref[...]
    y = jnp.dot(hsum, w_ref[...], preferred_element_type=jnp.float32) + b_ref[...]
    z = 0.5 * y * (1.0 + lax.erf(y * _SQRT_HALF))
    sq = jnp.sum(z * z, axis=-1, keepdims=True)
    o_ref[...] = z * lax.rsqrt(jnp.maximum(sq, 1e-12))


BR = 80


def _ffn(H, parts, Wp, bp):
    nblk = N // BR
    return pl.pallas_call(
        _ffn_body,
        out_shape=jax.ShapeDtypeStruct((N, D), jnp.float32),
        grid=(nblk,),
        in_specs=[
            pl.BlockSpec((BR, D), lambda i: (i, 0)),
            pl.BlockSpec((BR, D), lambda i: (i, 0)),
            pl.BlockSpec((BR, D), lambda i: (i + NPAD // BR, 0)),
            pl.BlockSpec((D, D), lambda i: (0, 0)),
            pl.BlockSpec((1, D), lambda i: (0, 0)),
        ],
        out_specs=pl.BlockSpec((BR, D), lambda i: (i, 0)),
    )(H, parts, parts, Wp, bp)


def _chunk_permute(a):
    return a.reshape(NW, CHUNK, K).swapaxes(1, 2).reshape(NW * K, CHUNK)


def kernel(H, edge_index, gamma, beta, moving_mean, moving_var, W, b):
    dst = edge_index[0].astype(jnp.int32)
    src = edge_index[1].astype(jnp.int32)
    pad = EPAD - E
    srcv = jnp.arange(pad, dtype=jnp.int32) % N
    src_r = _chunk_permute(jnp.concatenate([src, srcv]))
    padv = N + jnp.arange(pad, dtype=jnp.int32) % (NPAD - N)
    dst_r = _chunk_permute(jnp.concatenate([dst, padv]))

    parts = _make_sc_agg()(H, src_r, dst_r)

    scale = gamma * lax.rsqrt(moving_var + BN_EPS)
    shift = beta - moving_mean * scale
    Wp = scale[:, None] * W
    bp = (shift @ W + b).reshape(1, D)

    return _ffn(H, parts, Wp, bp)


# speedup vs baseline: 1.3802x; 1.3802x over previous
import functools

import jax
import jax.numpy as jnp
from jax import lax
from jax.experimental import pallas as pl
from jax.experimental.pallas import tpu as pltpu
from jax.experimental.pallas import tpu_sc as plsc

N = 10000
E = 320000
D = 128
BN_EPS = 1e-3

NC = 2
NS = 16
NW = NC * NS
CHUNK = 128
K = 8 * (-(-E // (NW * CHUNK * 8)))
EPW = K * CHUNK
EPAD = NW * EPW
NPAD = 10240
ZR = 16


def _sc_agg_body(
    h_hbm, srcr_hbm, dstr_hbm, out_hbm, sidx, didx, rows0, rows1, zbuf, acc, sem0, sem1
):
    c = lax.axis_index("c")
    s = lax.axis_index("s")
    w = c * NS + s

    zeros16 = jnp.zeros((16,), jnp.float32)
    for r in range(ZR):
        for q in range(D // 16):
            zbuf[r, pl.ds(q * 16, 16)] = zeros16

    rows_per_tile = NPAD // NS

    def zero_body(t, carry):
        pltpu.sync_copy(zbuf, acc.at[pl.ds(s * rows_per_tile + t * ZR, ZR)])
        return carry

    lax.fori_loop(0, rows_per_tile // ZR, zero_body, 0)

    plsc.subcore_barrier()

    NH = K // 2

    for h in range(2):
        pltpu.sync_copy(srcr_hbm.at[pl.ds(w * K + h * NH, NH)], sidx)
        pltpu.sync_copy(dstr_hbm.at[pl.ds(w * K + h * NH, NH)], didx)
        pltpu.async_copy(h_hbm.at[sidx.at[0]], rows0, sem0)

        def pair_body(t, carry):
            e0 = 2 * t
            pltpu.make_async_copy(h_hbm.at[sidx.at[e0]], rows0, sem0).wait()
            pltpu.async_copy(h_hbm.at[sidx.at[e0 + 1]], rows1, sem1)
            pltpu.sync_copy(rows0, acc.at[didx.at[e0]], add=True)
            nxt = jnp.minimum(e0 + 2, NH - 1)
            pltpu.make_async_copy(h_hbm.at[sidx.at[e0 + 1]], rows1, sem1).wait()
            pltpu.async_copy(h_hbm.at[sidx.at[nxt]], rows0, sem0)
            pltpu.sync_copy(rows1, acc.at[didx.at[e0 + 1]], add=True)
            return carry

        lax.fori_loop(0, NH // 2, pair_body, 0)
        pltpu.make_async_copy(h_hbm.at[sidx.at[NH - 1]], rows0, sem0).wait()

    plsc.subcore_barrier()

    pltpu.sync_copy(
        acc.at[pl.ds(s * rows_per_tile, rows_per_tile)],
        out_hbm.at[pl.ds(c * NPAD + s * rows_per_tile, rows_per_tile)],
    )


def _make_sc_agg():
    mesh = plsc.VectorSubcoreMesh(
        core_axis_name="c", subcore_axis_name="s", num_cores=NC, num_subcores=NS
    )
    return pl.kernel(
        _sc_agg_body,
        out_type=jax.ShapeDtypeStruct((NC * NPAD, D), jnp.float32),
        mesh=mesh,
        scratch_types=[
            pltpu.VMEM((K // 2, CHUNK), jnp.int32),
            pltpu.VMEM((K // 2, CHUNK), jnp.int32),
            pltpu.VMEM((CHUNK, D), jnp.float32),
            pltpu.VMEM((CHUNK, D), jnp.float32),
            pltpu.VMEM((ZR, D), jnp.float32),
            pltpu.VMEM_SHARED((NPAD, D), jnp.float32),
            pltpu.SemaphoreType.DMA,
            pltpu.SemaphoreType.DMA,
        ],
    )


_SQRT_HALF = 0.7071067811865476


def _ffn_body(h_ref, p0_ref, p1_ref, w_ref, b_ref, o_ref):
    hsum = h_ref[...] + p0_ref[0] + p1_ref[0]
    y = jnp.dot(hsum, w_ref[...], preferred_element_type=jnp.float32) + b_ref[...]
    z = 0.5 * y * (1.0 + lax.erf(y * _SQRT_HALF))
    sq = jnp.sum(z * z, axis=-1, keepdims=True)
    o_ref[...] = z * lax.rsqrt(jnp.maximum(sq, 1e-12))


BR = 1000


def _ffn(H, parts, Wp, bp):
    nblk = N // BR
    parts3 = parts.reshape(NC, NPAD, D)
    return pl.pallas_call(
        _ffn_body,
        out_shape=jax.ShapeDtypeStruct((N, D), jnp.float32),
        grid=(nblk,),
        in_specs=[
            pl.BlockSpec((BR, D), lambda i: (i, 0)),
            pl.BlockSpec((1, BR, D), lambda i: (0, i, 0)),
            pl.BlockSpec((1, BR, D), lambda i: (1, i, 0)),
            pl.BlockSpec((D, D), lambda i: (0, 0)),
            pl.BlockSpec((1, D), lambda i: (0, 0)),
        ],
        out_specs=pl.BlockSpec((BR, D), lambda i: (i, 0)),
    )(H, parts3, parts3, Wp, bp)


def kernel(H, edge_index, gamma, beta, moving_mean, moving_var, W, b):
    dst = edge_index[0].astype(jnp.int32)
    src = edge_index[1].astype(jnp.int32)
    pad = EPAD - E
    srcv = jnp.arange(pad, dtype=jnp.int32) % N
    src_r = jnp.concatenate([src, srcv]).reshape(NW * K, CHUNK)
    padv = N + jnp.arange(pad, dtype=jnp.int32) % (NPAD - N)
    dst_r = jnp.concatenate([dst, padv]).reshape(NW * K, CHUNK)

    parts = _make_sc_agg()(H, src_r, dst_r)

    scale = gamma * lax.rsqrt(moving_var + BN_EPS)
    shift = beta - moving_mean * scale
    Wp = scale[:, None] * W
    bp = (shift @ W + b).reshape(1, D)

    return _ffn(H, parts, Wp, bp)


# speedup vs baseline: 1.3886x; 1.0061x over previous
import functools

import jax
import jax.numpy as jnp
from jax import lax
from jax.experimental import pallas as pl
from jax.experimental.pallas import tpu as pltpu
from jax.experimental.pallas import tpu_sc as plsc

N = 10000
E = 320000
D = 128
BN_EPS = 1e-3

NC = 2
NS = 16
NW = NC * NS
CHUNK = 128
K = 8 * (-(-E // (NW * CHUNK * 8)))
EPW = K * CHUNK
EPAD = NW * EPW
CH2 = 64
K2 = EPW // CH2
NB = 4
NPAD = 10240
ZR = 16


def _sc_agg_body(
    h_hbm, srcr_hbm, dstr_hbm, out_hbm, sidx, didx,
    rows0, rows1, rows2, rows3, zbuf, acc,
    g0, g1, g2, g3, s0, s1, s2, s3,
):
    c = lax.axis_index("c")
    s = lax.axis_index("s")
    w = c * NS + s
    rows_l = (rows0, rows1, rows2, rows3)
    gsem_l = (g0, g1, g2, g3)
    ssem_l = (s0, s1, s2, s3)

    zeros16 = jnp.zeros((16,), jnp.float32)
    for r in range(ZR):
        for q in range(D // 16):
            zbuf[r, pl.ds(q * 16, 16)] = zeros16

    rows_per_tile = NPAD // NS

    def zero_body(t, carry):
        pltpu.sync_copy(zbuf, acc.at[pl.ds(s * rows_per_tile + t * ZR, ZR)])
        return carry

    lax.fori_loop(0, rows_per_tile // ZR, zero_body, 0)

    plsc.subcore_barrier()

    NH = K2 // 4

    for h in range(4):
        pltpu.sync_copy(srcr_hbm.at[pl.ds(w * K2 + h * NH, NH)], sidx)
        pltpu.sync_copy(dstr_hbm.at[pl.ds(w * K2 + h * NH, NH)], didx)
        for q in range(NB):
            pltpu.async_copy(h_hbm.at[sidx.at[q]], rows_l[q], gsem_l[q])

        def grp_body(t, carry):
            j0 = NB * t
            for q in range(NB):
                pltpu.make_async_copy(
                    h_hbm.at[sidx.at[j0 + q]], rows_l[q], gsem_l[q]
                ).wait()
                pltpu.async_copy(
                    rows_l[q], acc.at[didx.at[j0 + q]], ssem_l[q], add=True
                )
            for q in range(NB):
                pltpu.make_async_copy(
                    rows_l[q], acc.at[didx.at[j0 + q]], ssem_l[q]
                ).wait()
                nxt = jnp.minimum(j0 + NB + q, NH - 1)
                pltpu.async_copy(h_hbm.at[sidx.at[nxt]], rows_l[q], gsem_l[q])
            return carry

        lax.fori_loop(0, NH // NB, grp_body, 0)
        for q in range(NB):
            pltpu.make_async_copy(
                h_hbm.at[sidx.at[NH - 1]], rows_l[q], gsem_l[q]
            ).wait()

    plsc.subcore_barrier()

    pltpu.sync_copy(
        acc.at[pl.ds(s * rows_per_tile, rows_per_tile)],
        out_hbm.at[pl.ds(c * NPAD + s * rows_per_tile, rows_per_tile)],
    )


def _make_sc_agg():
    mesh = plsc.VectorSubcoreMesh(
        core_axis_name="c", subcore_axis_name="s", num_cores=NC, num_subcores=NS
    )
    return pl.kernel(
        _sc_agg_body,
        out_type=jax.ShapeDtypeStruct((NC * NPAD, D), jnp.float32),
        mesh=mesh,
        scratch_types=[
            pltpu.VMEM((K2 // 4, CH2), jnp.int32),
            pltpu.VMEM((K2 // 4, CH2), jnp.int32),
            pltpu.VMEM((CH2, D), jnp.float32),
            pltpu.VMEM((CH2, D), jnp.float32),
            pltpu.VMEM((CH2, D), jnp.float32),
            pltpu.VMEM((CH2, D), jnp.float32),
            pltpu.VMEM((ZR, D), jnp.float32),
            pltpu.VMEM_SHARED((NPAD, D), jnp.float32),
        ] + [pltpu.SemaphoreType.DMA] * 8,
    )


_SQRT_HALF = 0.7071067811865476


def _ffn_body(h_ref, p0_ref, p1_ref, w_ref, b_ref, o_ref):
    hsum = h_ref[...] + p0_ref[0] + p1_ref[0]
    y = jnp.dot(hsum, w_ref[...], preferred_element_type=jnp.float32) + b_ref[...]
    z = 0.5 * y * (1.0 + lax.erf(y * _SQRT_HALF))
    sq = jnp.sum(z * z, axis=-1, keepdims=True)
    o_ref[...] = z * lax.rsqrt(jnp.maximum(sq, 1e-12))


BR = 1000


def _ffn(H, parts, Wp, bp):
    nblk = N // BR
    parts3 = parts.reshape(NC, NPAD, D)
    return pl.pallas_call(
        _ffn_body,
        out_shape=jax.ShapeDtypeStruct((N, D), jnp.float32),
        grid=(nblk,),
        in_specs=[
            pl.BlockSpec((BR, D), lambda i: (i, 0)),
            pl.BlockSpec((1, BR, D), lambda i: (0, i, 0)),
            pl.BlockSpec((1, BR, D), lambda i: (1, i, 0)),
            pl.BlockSpec((D, D), lambda i: (0, 0)),
            pl.BlockSpec((1, D), lambda i: (0, 0)),
        ],
        out_specs=pl.BlockSpec((BR, D), lambda i: (i, 0)),
    )(H, parts3, parts3, Wp, bp)


def kernel(H, edge_index, gamma, beta, moving_mean, moving_var, W, b):
    dst = edge_index[0].astype(jnp.int32)
    src = edge_index[1].astype(jnp.int32)
    pad = EPAD - E
    srcv = jnp.arange(pad, dtype=jnp.int32) % N
    src_r = jnp.concatenate([src, srcv]).reshape(NW * K2, CH2)
    padv = N + jnp.arange(pad, dtype=jnp.int32) % (NPAD - N)
    dst_r = jnp.concatenate([dst, padv]).reshape(NW * K2, CH2)

    parts = _make_sc_agg()(H, src_r, dst_r)

    scale = gamma * lax.rsqrt(moving_var + BN_EPS)
    shift = beta - moving_mean * scale
    Wp = scale[:, None] * W
    bp = (shift @ W + b).reshape(1, D)

    return _ffn(H, parts, Wp, bp)


# speedup vs baseline: 1.3926x; 1.0029x over previous
import functools

import jax
import jax.numpy as jnp
from jax import lax
from jax.experimental import pallas as pl
from jax.experimental.pallas import tpu as pltpu
from jax.experimental.pallas import tpu_sc as plsc

N = 10000
E = 320000
D = 128
BN_EPS = 1e-3

NC = 2
NS = 16
NW = NC * NS
CHUNK = 128
K = 8 * (-(-E // (NW * CHUNK * 8)))
EPW = K * CHUNK
EPAD = NW * EPW
CH2 = 64
K2 = EPW // CH2
TOTROWS = E // CH2
NB = 4
NPAD = 10240
ZR = 16


def _sc_agg_body(
    h_hbm, srcr_hbm, dstr_hbm, out_hbm, sidx, didx,
    rows0, rows1, rows2, rows3, zbuf, acc,
    g0, g1, g2, g3, s0, s1, s2, s3,
):
    c = lax.axis_index("c")
    s = lax.axis_index("s")
    w = c * NS + s
    rows_l = (rows0, rows1, rows2, rows3)
    gsem_l = (g0, g1, g2, g3)
    ssem_l = (s0, s1, s2, s3)

    zeros16 = jnp.zeros((16,), jnp.float32)
    for r in range(ZR):
        for q in range(D // 16):
            zbuf[r, pl.ds(q * 16, 16)] = zeros16

    rows_per_tile = NPAD // NS

    def zero_body(t, carry):
        pltpu.sync_copy(zbuf, acc.at[pl.ds(s * rows_per_tile + t * ZR, ZR)])
        return carry

    lax.fori_loop(0, rows_per_tile // ZR, zero_body, 0)

    plsc.subcore_barrier()

    NH = K2 // 4
    start_row = w * K2
    nsub = jnp.clip(TOTROWS - start_row, 0, K2)

    for h in range(4):
        off = pl.multiple_of(jnp.minimum(start_row + h * NH, TOTROWS - NH), 8)
        cnt = jnp.clip(nsub - h * NH, 0, NH)
        pltpu.sync_copy(srcr_hbm.at[pl.ds(off, NH)], sidx)
        pltpu.sync_copy(dstr_hbm.at[pl.ds(off, NH)], didx)
        for q in range(NB):
            pltpu.async_copy(h_hbm.at[sidx.at[q]], rows_l[q], gsem_l[q])

        def grp_body(t, carry):
            j0 = NB * t
            for q in range(NB):
                pltpu.make_async_copy(
                    h_hbm.at[sidx.at[j0 + q]], rows_l[q], gsem_l[q]
                ).wait()
                pltpu.async_copy(
                    rows_l[q], acc.at[didx.at[j0 + q]], ssem_l[q], add=True
                )
            for q in range(NB):
                pltpu.make_async_copy(
                    rows_l[q], acc.at[didx.at[j0 + q]], ssem_l[q]
                ).wait()
                nxt = jnp.minimum(j0 + NB + q, NH - 1)
                pltpu.async_copy(h_hbm.at[sidx.at[nxt]], rows_l[q], gsem_l[q])
            return carry

        lax.fori_loop(0, cnt // NB, grp_body, 0)
        for q in range(NB):
            pltpu.make_async_copy(
                h_hbm.at[sidx.at[NH - 1]], rows_l[q], gsem_l[q]
            ).wait()

    plsc.subcore_barrier()

    pltpu.sync_copy(
        acc.at[pl.ds(s * rows_per_tile, rows_per_tile)],
        out_hbm.at[pl.ds(c * NPAD + s * rows_per_tile, rows_per_tile)],
    )


def _make_sc_agg():
    mesh = plsc.VectorSubcoreMesh(
        core_axis_name="c", subcore_axis_name="s", num_cores=NC, num_subcores=NS
    )
    return pl.kernel(
        _sc_agg_body,
        out_type=jax.ShapeDtypeStruct((NC * NPAD, D), jnp.float32),
        mesh=mesh,
        scratch_types=[
            pltpu.VMEM((K2 // 4, CH2), jnp.int32),
            pltpu.VMEM((K2 // 4, CH2), jnp.int32),
            pltpu.VMEM((CH2, D), jnp.float32),
            pltpu.VMEM((CH2, D), jnp.float32),
            pltpu.VMEM((CH2, D), jnp.float32),
            pltpu.VMEM((CH2, D), jnp.float32),
            pltpu.VMEM((ZR, D), jnp.float32),
            pltpu.VMEM_SHARED((NPAD, D), jnp.float32),
        ] + [pltpu.SemaphoreType.DMA] * 8,
    )


_SQRT_HALF = 0.7071067811865476


def _ffn_body(h_ref, p0_ref, p1_ref, w_ref, b_ref, o_ref):
    hsum = h_ref[...] + p0_ref[0] + p1_ref[0]
    y = jnp.dot(hsum, w_ref[...], preferred_element_type=jnp.float32) + b_ref[...]
    z = 0.5 * y * (1.0 + lax.erf(y * _SQRT_HALF))
    sq = jnp.sum(z * z, axis=-1, keepdims=True)
    o_ref[...] = z * lax.rsqrt(jnp.maximum(sq, 1e-12))


BR = 1000


def _ffn(H, parts, Wp, bp):
    nblk = N // BR
    parts3 = parts.reshape(NC, NPAD, D)
    return pl.pallas_call(
        _ffn_body,
        out_shape=jax.ShapeDtypeStruct((N, D), jnp.float32),
        grid=(nblk,),
        in_specs=[
            pl.BlockSpec((BR, D), lambda i: (i, 0)),
            pl.BlockSpec((1, BR, D), lambda i: (0, i, 0)),
            pl.BlockSpec((1, BR, D), lambda i: (1, i, 0)),
            pl.BlockSpec((D, D), lambda i: (0, 0)),
            pl.BlockSpec((1, D), lambda i: (0, 0)),
        ],
        out_specs=pl.BlockSpec((BR, D), lambda i: (i, 0)),
    )(H, parts3, parts3, Wp, bp)


def kernel(H, edge_index, gamma, beta, moving_mean, moving_var, W, b):
    dst = edge_index[0].astype(jnp.int32)
    src = edge_index[1].astype(jnp.int32)
    src_r = src.reshape(TOTROWS, CH2)
    dst_r = dst.reshape(TOTROWS, CH2)

    parts = _make_sc_agg()(H, src_r, dst_r)

    scale = gamma * lax.rsqrt(moving_var + BN_EPS)
    shift = beta - moving_mean * scale
    Wp = scale[:, None] * W
    bp = (shift @ W + b).reshape(1, D)

    return _ffn(H, parts, Wp, bp)


# speedup vs baseline: 1.4148x; 1.0159x over previous
import functools

import jax
import jax.numpy as jnp
from jax import lax
from jax.experimental import pallas as pl
from jax.experimental.pallas import tpu as pltpu
from jax.experimental.pallas import tpu_sc as plsc

N = 10000
E = 320000
D = 128
BN_EPS = 1e-3

NC = 2
NS = 16
NW = NC * NS
CHUNK = 128
K = 8 * (-(-E // (NW * CHUNK * 8)))
EPW = K * CHUNK
EPAD = NW * EPW
CH2 = 64
K2 = EPW // CH2
TOTROWS = E // CH2
NB = 4
NPAD = 10240
ZR = 16


def _sc_agg_body(
    h_hbm, srcr_hbm, dstr_hbm, out_hbm, sidx, didx,
    rows0, rows1, rows2, rows3, zbuf, acc,
    g0, g1, g2, g3, s0, s1, s2, s3,
):
    c = lax.axis_index("c")
    s = lax.axis_index("s")
    w = c * NS + s
    rows_l = (rows0, rows1, rows2, rows3)
    gsem_l = (g0, g1, g2, g3)
    ssem_l = (s0, s1, s2, s3)

    zeros16 = jnp.zeros((16,), jnp.float32)
    for r in range(ZR):
        for q in range(D // 16):
            zbuf[r, pl.ds(q * 16, 16)] = zeros16

    rows_per_tile = NPAD // NS

    def zero_body(t, carry):
        pltpu.sync_copy(zbuf, acc.at[pl.ds(s * rows_per_tile + t * ZR, ZR)])
        return carry

    lax.fori_loop(0, rows_per_tile // ZR, zero_body, 0)

    plsc.subcore_barrier()

    NH = K2 // 4
    start_row = w * K2
    nsub = jnp.clip(TOTROWS - start_row, 0, K2)

    for h in range(4):
        off = pl.multiple_of(jnp.minimum(start_row + h * NH, TOTROWS - NH), 8)
        cnt = jnp.clip(nsub - h * NH, 0, NH)
        pltpu.sync_copy(srcr_hbm.at[pl.ds(off, NH)], sidx)
        pltpu.sync_copy(dstr_hbm.at[pl.ds(off, NH)], didx)
        for q in range(NB):
            pltpu.async_copy(h_hbm.at[sidx.at[q]], rows_l[q], gsem_l[q])

        def grp_body(t, carry):
            j0 = NB * t
            for q in range(NB):
                pltpu.make_async_copy(
                    h_hbm.at[sidx.at[j0 + q]], rows_l[q], gsem_l[q]
                ).wait()
                pltpu.async_copy(
                    rows_l[q], acc.at[didx.at[j0 + q]], ssem_l[q], add=True
                )
            for q in range(NB):
                pltpu.make_async_copy(
                    rows_l[q], acc.at[didx.at[j0 + q]], ssem_l[q]
                ).wait()
                nxt = jnp.minimum(j0 + NB + q, NH - 1)
                pltpu.async_copy(h_hbm.at[sidx.at[nxt]], rows_l[q], gsem_l[q])
            return carry

        lax.fori_loop(0, cnt // NB, grp_body, 0)
        for q in range(NB):
            pltpu.make_async_copy(
                h_hbm.at[sidx.at[NH - 1]], rows_l[q], gsem_l[q]
            ).wait()

    plsc.subcore_barrier()

    pltpu.sync_copy(
        acc.at[pl.ds(s * rows_per_tile, rows_per_tile)],
        out_hbm.at[pl.ds(c * NPAD + s * rows_per_tile, rows_per_tile)],
    )


def _make_sc_agg():
    mesh = plsc.VectorSubcoreMesh(
        core_axis_name="c", subcore_axis_name="s", num_cores=NC, num_subcores=NS
    )
    return pl.kernel(
        _sc_agg_body,
        out_type=jax.ShapeDtypeStruct((NC * NPAD, D), jnp.float32),
        mesh=mesh,
        scratch_types=[
            pltpu.VMEM((K2 // 4, CH2), jnp.int32),
            pltpu.VMEM((K2 // 4, CH2), jnp.int32),
            pltpu.VMEM((CH2, D), jnp.float32),
            pltpu.VMEM((CH2, D), jnp.float32),
            pltpu.VMEM((CH2, D), jnp.float32),
            pltpu.VMEM((CH2, D), jnp.float32),
            pltpu.VMEM((ZR, D), jnp.float32),
            pltpu.VMEM_SHARED((NPAD, D), jnp.float32),
        ] + [pltpu.SemaphoreType.DMA] * 8,
    )


_SQRT_HALF = 0.7071067811865476


def _ffn_body(h_ref, p0_ref, p1_ref, w_ref, b_ref, o_ref):
    hsum = h_ref[...] + p0_ref[0] + p1_ref[0]
    y = jnp.dot(hsum, w_ref[...], preferred_element_type=jnp.float32) + b_ref[...]
    z = 0.5 * y * (1.0 + lax.erf(y * _SQRT_HALF))
    sq = jnp.sum(z * z, axis=-1, keepdims=True)
    o_ref[...] = z * lax.rsqrt(jnp.maximum(sq, 1e-12))


BR = 2000


def _ffn(H, parts, Wp, bp):
    nblk = N // BR
    parts3 = parts.reshape(NC, NPAD, D)
    return pl.pallas_call(
        _ffn_body,
        out_shape=jax.ShapeDtypeStruct((N, D), jnp.float32),
        grid=(nblk,),
        in_specs=[
            pl.BlockSpec((BR, D), lambda i: (i, 0)),
            pl.BlockSpec((1, BR, D), lambda i: (0, i, 0)),
            pl.BlockSpec((1, BR, D), lambda i: (1, i, 0)),
            pl.BlockSpec((D, D), lambda i: (0, 0)),
            pl.BlockSpec((1, D), lambda i: (0, 0)),
        ],
        out_specs=pl.BlockSpec((BR, D), lambda i: (i, 0)),
    )(H, parts3, parts3, Wp, bp)


def kernel(H, edge_index, gamma, beta, moving_mean, moving_var, W, b):
    dst = edge_index[0].astype(jnp.int32)
    src = edge_index[1].astype(jnp.int32)
    src_r = src.reshape(TOTROWS, CH2)
    dst_r = dst.reshape(TOTROWS, CH2)

    parts = _make_sc_agg()(H, src_r, dst_r)

    scale = gamma * lax.rsqrt(moving_var + BN_EPS)
    shift = beta - moving_mean * scale
    Wp = scale[:, None] * W
    bp = (shift @ W + b).reshape(1, D)

    return _ffn(H, parts, Wp, bp)
